# baseline (speedup 1.0000x reference)
import jax
import jax.numpy as jnp
from jax.experimental import pallas as pl
from jax.experimental.pallas import tpu as pltpu

_K = 8


def _gcn_body(data_ref, adj_ref, w_ref, b_ref, out_ref, mask_ref, cnt_ref,
              xw_ref):
    k = pl.program_id(0)
    bn = adj_ref.shape[0]
    n = adj_ref.shape[1]
    f = w_ref.shape[0]

    mb = (adj_ref[...] != 0.0).astype(jnp.bfloat16)
    mask_ref[pl.ds(k * bn, bn), :] = mb
    ones_col = jnp.ones((bn, 1), jnp.bfloat16)
    partial = jax.lax.dot_general(
        mb, ones_col, (((0,), (0,)), ((), ())),
        preferred_element_type=jnp.float32)

    @pl.when(k == 0)
    def _init():
        cnt_ref[...] = partial
        x = data_ref[...].reshape(2 * n, f)
        xw_ref[...] = jax.lax.dot_general(
            x, w_ref[...], (((1,), (1,)), ((), ())),
            preferred_element_type=jnp.float32,
            precision=jax.lax.Precision.HIGHEST)

    @pl.when(k > 0)
    def _acc():
        cnt_ref[...] = cnt_ref[...] + partial

    @pl.when(k == _K - 1)
    def _finalize():
        cnt = cnt_ref[...]
        nnz = jnp.sum(cnt)
        pad = jnp.float32(n) * jnp.float32(n) - nnz
        row_ids = jax.lax.broadcasted_iota(jnp.int32, (n, 1), 0)
        is_row0 = (row_ids == 0).astype(jnp.float32)
        deg = 2.0 * (cnt + pad * is_row0) + 1.0
        dis = jax.lax.rsqrt(deg)

        xw0 = xw_ref[pl.ds(0, n), :]
        xw1 = xw_ref[pl.ds(n, n), :]
        v = dis * xw0
        v_hi = v.astype(jnp.bfloat16)
        v_lo = (v - v_hi.astype(jnp.float32)).astype(jnp.bfloat16)
        mask = mask_ref[...]
        dims = (((0,), (0,)), ((), ()))
        s = (jax.lax.dot_general(mask, v_hi, dims,
                                 preferred_element_type=jnp.float32)
             + jax.lax.dot_general(mask, v_lo, dims,
                                   preferred_element_type=jnp.float32))
        s = s + is_row0 * (pad * v[0:1, :])

        b_row = b_ref[...]
        out_ref[0] = (2.0 * dis) * s + (dis * dis) * xw0 + b_row
        out_ref[1] = xw1 + b_row


def kernel(data, adj, W, b):
    batch, n, f = data.shape
    bn = n // _K
    return pl.pallas_call(
        _gcn_body,
        grid=(_K,),
        in_specs=[
            pl.BlockSpec((batch, n, f), lambda k: (0, 0, 0)),
            pl.BlockSpec((bn, n), lambda k: (k, 0)),
            pl.BlockSpec((f, f), lambda k: (0, 0)),
            pl.BlockSpec((1, f), lambda k: (0, 0)),
        ],
        out_specs=pl.BlockSpec((batch, n, f), lambda k: (0, 0, 0)),
        out_shape=jax.ShapeDtypeStruct((batch, n, f), data.dtype),
        scratch_shapes=[
            pltpu.VMEM((n, n), jnp.bfloat16),
            pltpu.VMEM((n, 1), jnp.float32),
            pltpu.VMEM((2 * n, f), jnp.float32),
        ],
    )(data, adj, W, b.reshape(1, f))


# speedup vs baseline: 1.2736x; 1.2736x over previous
import jax
import jax.numpy as jnp
from jax.experimental import pallas as pl
from jax.experimental.pallas import tpu as pltpu

_K = 2


def _gcn_body(data_ref, adj_ref, w_ref, b_ref, out_ref, mask_ref, cnt_ref,
              xw_ref):
    k = pl.program_id(0)
    bn = adj_ref.shape[0]
    n = adj_ref.shape[1]
    f = w_ref.shape[0]

    mb = (adj_ref[...] != 0.0).astype(jnp.bfloat16)
    mask_ref[pl.ds(k * bn, bn), :] = mb
    ones_col = jnp.ones((bn, 1), jnp.bfloat16)
    partial = jax.lax.dot_general(
        mb, ones_col, (((0,), (0,)), ((), ())),
        preferred_element_type=jnp.float32)

    @pl.when(k == 0)
    def _init():
        cnt_ref[...] = partial
        x = data_ref[...].reshape(2 * n, f)
        xw_ref[...] = jax.lax.dot_general(
            x, w_ref[...], (((1,), (1,)), ((), ())),
            preferred_element_type=jnp.float32,
            precision=jax.lax.Precision.HIGHEST)

    @pl.when(k > 0)
    def _acc():
        cnt_ref[...] = cnt_ref[...] + partial

    @pl.when(k == _K - 1)
    def _finalize():
        cnt = cnt_ref[...]
        nnz = jnp.sum(cnt)
        pad = jnp.float32(n) * jnp.float32(n) - nnz
        row_ids = jax.lax.broadcasted_iota(jnp.int32, (n, 1), 0)
        is_row0 = (row_ids == 0).astype(jnp.float32)
        deg = 2.0 * (cnt + pad * is_row0) + 1.0
        dis = jax.lax.rsqrt(deg)

        xw0 = xw_ref[pl.ds(0, n), :]
        xw1 = xw_ref[pl.ds(n, n), :]
        v = dis * xw0
        v_hi = v.astype(jnp.bfloat16)
        v_lo = (v - v_hi.astype(jnp.float32)).astype(jnp.bfloat16)
        mask = mask_ref[...]
        dims = (((0,), (0,)), ((), ()))
        s = (jax.lax.dot_general(mask, v_hi, dims,
                                 preferred_element_type=jnp.float32)
             + jax.lax.dot_general(mask, v_lo, dims,
                                   preferred_element_type=jnp.float32))
        s = s + is_row0 * (pad * v[0:1, :])

        b_row = b_ref[...]
        out_ref[0] = (2.0 * dis) * s + (dis * dis) * xw0 + b_row
        out_ref[1] = xw1 + b_row


def kernel(data, adj, W, b):
    batch, n, f = data.shape
    bn = n // _K
    return pl.pallas_call(
        _gcn_body,
        grid=(_K,),
        in_specs=[
            pl.BlockSpec((batch, n, f), lambda k: (0, 0, 0)),
            pl.BlockSpec((bn, n), lambda k: (k, 0)),
            pl.BlockSpec((f, f), lambda k: (0, 0)),
            pl.BlockSpec((1, f), lambda k: (0, 0)),
        ],
        out_specs=pl.BlockSpec((batch, n, f), lambda k: (0, 0, 0)),
        out_shape=jax.ShapeDtypeStruct((batch, n, f), data.dtype),
        scratch_shapes=[
            pltpu.VMEM((n, n), jnp.bfloat16),
            pltpu.VMEM((n, 1), jnp.float32),
            pltpu.VMEM((2 * n, f), jnp.float32),
        ],
    )(data, adj, W, b.reshape(1, f))


# speedup vs baseline: 1.2896x; 1.0126x over previous
import jax
import jax.numpy as jnp
from jax.experimental import pallas as pl


def _gcn_body(data_ref, adj_ref, w_ref, b_ref, out_ref):
    n = adj_ref.shape[0]
    f = w_ref.shape[0]
    adj = adj_ref[...]
    mask = (adj != 0.0).astype(jnp.bfloat16)

    ones_col = jnp.ones((n, 1), jnp.bfloat16)
    cnt = jax.lax.dot_general(
        mask, ones_col, (((0,), (0,)), ((), ())),
        preferred_element_type=jnp.float32)
    nnz = jnp.sum(cnt)
    pad = jnp.float32(n) * jnp.float32(n) - nnz

    row_ids = jax.lax.broadcasted_iota(jnp.int32, (n, 1), 0)
    is_row0 = (row_ids == 0).astype(jnp.float32)
    cnt = cnt + pad * is_row0
    deg = 2.0 * cnt + 1.0
    dis = jax.lax.rsqrt(deg)

    x = data_ref[...].reshape(2 * n, f)
    xw = jax.lax.dot_general(
        x, w_ref[...], (((1,), (1,)), ((), ())),
        preferred_element_type=jnp.float32)
    xw0 = xw[:n]
    xw1 = xw[n:]

    v = dis * xw0
    v_hi = v.astype(jnp.bfloat16)
    v_lo = (v - v_hi.astype(jnp.float32)).astype(jnp.bfloat16)
    dims = (((0,), (0,)), ((), ()))
    s = (jax.lax.dot_general(mask, v_hi, dims,
                             preferred_element_type=jnp.float32)
         + jax.lax.dot_general(mask, v_lo, dims,
                               preferred_element_type=jnp.float32))
    s = s + is_row0 * (pad * v[0:1, :])

    b_row = b_ref[...]
    out_ref[0] = (2.0 * dis) * s + (dis * dis) * xw0 + b_row
    out_ref[1] = xw1 + b_row


def kernel(data, adj, W, b):
    batch, n, f = data.shape
    return pl.pallas_call(
        _gcn_body,
        out_shape=jax.ShapeDtypeStruct((batch, n, f), data.dtype),
    )(data, adj, W, b.reshape(1, f))
